# baseline (speedup 1.0000x reference)
import functools

import jax
import jax.numpy as jnp
from jax import lax
from jax.experimental import pallas as pl
from jax.experimental.pallas import tpu as pltpu
from jax.experimental.pallas import tpu_sc as plsc

_NC = 2
_NS = 16
_CH = 104
_RCH = 80


def _make_agg(N, ept, D):
    nch = ept // _CH
    assert ept % _CH == 0 and nch % 2 == 1 and N % _RCH == 0
    nrch = N // _RCH
    NP = N + _NS
    mesh = plsc.VectorSubcoreMesh(core_axis_name="c", subcore_axis_name="s")

    @functools.partial(
        pl.kernel,
        mesh=mesh,
        out_type=jax.ShapeDtypeStruct((_NC, N, D), jnp.float32),
        scratch_types=[
            pltpu.VMEM((ept,), jnp.int32),
            pltpu.VMEM((nch, _CH), jnp.int32),
            pltpu.VMEM((_CH, D), jnp.float32),
            pltpu.VMEM((_CH, D), jnp.float32),
            pltpu.VMEM_SHARED((NP, D), jnp.float32),
            pltpu.SemaphoreType.DMA,
            pltpu.SemaphoreType.DMA,
            pltpu.SemaphoreType.DMA,
            pltpu.SemaphoreType.DMA,
        ],
    )
    def agg(x_hbm, src_hbm, dst_hbm, out_hbm, src_v, dst_v, rows0, rows1,
            acc_sh, sem0, sem1, ss0, ss1):
        c = lax.axis_index("c")
        s = lax.axis_index("s")
        wid = s * _NC + c

        idx_cp0 = pltpu.async_copy(src_hbm.at[pl.ds(wid * ept, ept)],
                                   src_v, sem0)
        idx_cp1 = pltpu.async_copy(dst_hbm.at[wid], dst_v, sem1)

        zero16 = jnp.zeros((16,), jnp.float32)

        def zfill(i, _):
            rows0[i // (D // 16), pl.ds((i % (D // 16)) * 16, 16)] = zero16
            return 0

        lax.fori_loop(0, _CH * D // 16, zfill, 0)

        def zcopy(j, _):
            k = s + j * _NS

            @pl.when(k < nrch)
            def _():
                pltpu.sync_copy(rows0.at[pl.ds(0, _RCH)],
                                acc_sh.at[pl.ds(k * _RCH, _RCH)])

            return 0

        lax.fori_loop(0, (nrch + _NS - 1) // _NS, zcopy, 0)
        idx_cp0.wait()
        idx_cp1.wait()
        plsc.subcore_barrier()

        def sidx(j):
            return src_v.at[pl.ds(j * _CH, _CH)]

        pltpu.async_copy(x_hbm.at[sidx(0)], rows0, sem0)
        pltpu.async_copy(x_hbm.at[sidx(1)], rows1, sem1)

        def pair(i, _):
            j = 2 * i
            pltpu.make_async_copy(x_hbm.at[sidx(j)], rows0, sem0).wait()
            pltpu.async_copy(rows0, acc_sh.at[dst_v.at[j]], ss0, add=True)
            pltpu.make_async_copy(x_hbm.at[sidx(j + 1)], rows1, sem1).wait()
            pltpu.async_copy(rows1, acc_sh.at[dst_v.at[j + 1]], ss1, add=True)
            pltpu.make_async_copy(rows0, acc_sh.at[dst_v.at[j]], ss0).wait()
            pltpu.async_copy(x_hbm.at[sidx(j + 2)], rows0, sem0)
            pltpu.make_async_copy(rows1, acc_sh.at[dst_v.at[j + 1]],
                                  ss1).wait()

            @pl.when(j + 3 < nch)
            def _():
                pltpu.async_copy(x_hbm.at[sidx(j + 3)], rows1, sem1)

            return 0

        lax.fori_loop(0, (nch - 1) // 2, pair, 0)
        pltpu.make_async_copy(x_hbm.at[sidx(nch - 1)], rows0, sem0).wait()
        pltpu.sync_copy(rows0, acc_sh.at[dst_v.at[nch - 1]], add=True)
        plsc.subcore_barrier()

        def dcopy(j, _):
            k = s + j * _NS

            @pl.when(k < nrch)
            def _():
                rr = k * _RCH
                pltpu.sync_copy(acc_sh.at[pl.ds(rr, _RCH)],
                                rows0.at[pl.ds(0, _RCH)])
                pltpu.sync_copy(rows0.at[pl.ds(0, _RCH)],
                                out_hbm.at[c, pl.ds(rr, _RCH)])

            return 0

        lax.fori_loop(0, (nrch + _NS - 1) // _NS, dcopy, 0)

    return agg


def _final(x, p0, p1, deg_inv, W):
    N, D = x.shape
    DO = W.shape[1]
    B = 2000
    assert N % B == 0

    def body(x_ref, p0_ref, p1_ref, dinv_ref, w_ref, o_ref):
        az = x_ref[...] + p0_ref[...] + p1_ref[...]
        azw = jnp.dot(az, w_ref[...], preferred_element_type=jnp.float32)
        o_ref[...] = jnp.tanh(dinv_ref[...] * azw)

    return pl.pallas_call(
        body,
        grid=(N // B,),
        in_specs=[
            pl.BlockSpec((B, D), lambda i: (i, 0)),
            pl.BlockSpec((B, D), lambda i: (i, 0)),
            pl.BlockSpec((B, D), lambda i: (i, 0)),
            pl.BlockSpec((B, 1), lambda i: (i, 0)),
            pl.BlockSpec((D, DO), lambda i: (0, 0)),
        ],
        out_specs=pl.BlockSpec((B, DO), lambda i: (i, 0)),
        out_shape=jax.ShapeDtypeStruct((N, DO), jnp.float32),
    )(x, p0, p1, deg_inv.reshape(N, 1), W)


def kernel(x, edge_index, deg_inv, W):
    N, D = x.shape
    E = edge_index.shape[1]
    NW = _NC * _NS
    ept_raw = E // NW
    nch = -(-ept_raw // _CH)
    if nch % 2 == 0:
        nch += 1
    ept = nch * _CH
    pad = ept - ept_raw

    srcs = jnp.pad(edge_index[1].reshape(NW, ept_raw), ((0, 0), (0, pad)))
    padcol = N + jnp.arange(NW, dtype=jnp.int32)[:, None] // _NC
    dsts = jnp.concatenate(
        [edge_index[0].reshape(NW, ept_raw),
         jnp.broadcast_to(padcol, (NW, pad))], axis=1)
    dst3 = dsts.reshape(NW, nch, _CH)

    parts = _make_agg(N, ept, D)(x, srcs.reshape(-1), dst3)
    return _final(x, parts[0], parts[1], deg_inv, W)


# speedup vs baseline: 1.0330x; 1.0330x over previous
import functools

import jax
import jax.numpy as jnp
from jax import lax
from jax.experimental import pallas as pl
from jax.experimental.pallas import tpu as pltpu
from jax.experimental.pallas import tpu_sc as plsc

_NC = 2
_NS = 16
_CH = 96
_RCH = 80


def _make_agg(N, ept, D):
    nch = ept // _CH
    assert ept % _CH == 0 and nch % 2 == 1 and N % _RCH == 0
    nrch = N // _RCH
    NP = N + _NS
    mesh = plsc.VectorSubcoreMesh(core_axis_name="c", subcore_axis_name="s")

    @functools.partial(
        pl.kernel,
        mesh=mesh,
        out_type=jax.ShapeDtypeStruct((_NC, N, D), jnp.float32),
        scratch_types=[
            pltpu.VMEM((ept,), jnp.int32),
            pltpu.VMEM((nch, _CH), jnp.int32),
            pltpu.VMEM((_CH, D), jnp.float32),
            pltpu.VMEM((_CH, D), jnp.float32),
            pltpu.VMEM_SHARED((NP, D), jnp.float32),
            pltpu.SemaphoreType.DMA,
            pltpu.SemaphoreType.DMA,
            pltpu.SemaphoreType.DMA,
            pltpu.SemaphoreType.DMA,
        ],
    )
    def agg(x_hbm, src_hbm, dst_hbm, out_hbm, src_v, dst_v, rows0, rows1,
            acc_sh, sem0, sem1, ss0, ss1):
        c = lax.axis_index("c")
        s = lax.axis_index("s")
        wid = s * _NC + c

        idx_cp0 = pltpu.async_copy(src_hbm.at[pl.ds(wid * ept, ept)],
                                   src_v, sem0)
        idx_cp1 = pltpu.async_copy(dst_hbm.at[wid], dst_v, sem1)

        zero16 = jnp.zeros((16,), jnp.float32)

        def zfill(i, _):
            rows0[i // (D // 16), pl.ds((i % (D // 16)) * 16, 16)] = zero16
            return 0

        lax.fori_loop(0, _CH * D // 16, zfill, 0)

        def zcopy(j, _):
            k = s + j * _NS

            @pl.when(k < nrch)
            def _():
                pltpu.sync_copy(rows0.at[pl.ds(0, _RCH)],
                                acc_sh.at[pl.ds(k * _RCH, _RCH)])

            return 0

        lax.fori_loop(0, (nrch + _NS - 1) // _NS, zcopy, 0)
        idx_cp0.wait()
        idx_cp1.wait()
        plsc.subcore_barrier()

        def sidx(j):
            return src_v.at[pl.ds(j * _CH, _CH)]

        pltpu.async_copy(x_hbm.at[sidx(0)], rows0, sem0)
        pltpu.async_copy(x_hbm.at[sidx(1)], rows1, sem1)

        def pair(i, _):
            j = 2 * i
            pltpu.make_async_copy(x_hbm.at[sidx(j)], rows0, sem0).wait()
            pltpu.async_copy(rows0, acc_sh.at[dst_v.at[j]], ss0, add=True)
            pltpu.make_async_copy(x_hbm.at[sidx(j + 1)], rows1, sem1).wait()
            pltpu.async_copy(rows1, acc_sh.at[dst_v.at[j + 1]], ss1, add=True)
            pltpu.make_async_copy(rows0, acc_sh.at[dst_v.at[j]], ss0).wait()
            pltpu.async_copy(x_hbm.at[sidx(j + 2)], rows0, sem0)
            pltpu.make_async_copy(rows1, acc_sh.at[dst_v.at[j + 1]],
                                  ss1).wait()

            @pl.when(j + 3 < nch)
            def _():
                pltpu.async_copy(x_hbm.at[sidx(j + 3)], rows1, sem1)

            return 0

        lax.fori_loop(0, (nch - 1) // 2, pair, 0)
        pltpu.make_async_copy(x_hbm.at[sidx(nch - 1)], rows0, sem0).wait()
        pltpu.sync_copy(rows0, acc_sh.at[dst_v.at[nch - 1]], add=True)
        plsc.subcore_barrier()

        def dcopy(j, _):
            k = s + j * _NS

            @pl.when(k < nrch)
            def _():
                rr = k * _RCH
                pltpu.sync_copy(acc_sh.at[pl.ds(rr, _RCH)],
                                rows0.at[pl.ds(0, _RCH)])
                pltpu.sync_copy(rows0.at[pl.ds(0, _RCH)],
                                out_hbm.at[c, pl.ds(rr, _RCH)])

            return 0

        lax.fori_loop(0, (nrch + _NS - 1) // _NS, dcopy, 0)

    return agg


def _final(x, p0, p1, deg_inv, W):
    N, D = x.shape
    DO = W.shape[1]
    B = 2000
    assert N % B == 0

    def body(x_ref, p0_ref, p1_ref, dinv_ref, w_ref, o_ref):
        az = x_ref[...] + p0_ref[...] + p1_ref[...]
        azw = jnp.dot(az, w_ref[...], preferred_element_type=jnp.float32)
        o_ref[...] = jnp.tanh(dinv_ref[...] * azw)

    return pl.pallas_call(
        body,
        grid=(N // B,),
        in_specs=[
            pl.BlockSpec((B, D), lambda i: (i, 0)),
            pl.BlockSpec((B, D), lambda i: (i, 0)),
            pl.BlockSpec((B, D), lambda i: (i, 0)),
            pl.BlockSpec((B, 1), lambda i: (i, 0)),
            pl.BlockSpec((D, DO), lambda i: (0, 0)),
        ],
        out_specs=pl.BlockSpec((B, DO), lambda i: (i, 0)),
        out_shape=jax.ShapeDtypeStruct((N, DO), jnp.float32),
    )(x, p0, p1, deg_inv.reshape(N, 1), W)


def kernel(x, edge_index, deg_inv, W):
    N, D = x.shape
    E = edge_index.shape[1]
    NW = _NC * _NS
    ept_raw = E // NW
    nch = -(-ept_raw // _CH)
    if nch % 2 == 0:
        nch += 1
    ept = nch * _CH
    pad = ept - ept_raw

    srcs = jnp.pad(edge_index[1].reshape(NW, ept_raw), ((0, 0), (0, pad)))
    padcol = N + jnp.arange(NW, dtype=jnp.int32)[:, None] // _NC
    dsts = jnp.concatenate(
        [edge_index[0].reshape(NW, ept_raw),
         jnp.broadcast_to(padcol, (NW, pad))], axis=1)
    dst3 = dsts.reshape(NW, nch, _CH)

    parts = _make_agg(N, ept, D)(x, srcs.reshape(-1), dst3)
    return _final(x, parts[0], parts[1], deg_inv, W)


# speedup vs baseline: 1.4888x; 1.4412x over previous
import functools

import jax
import jax.numpy as jnp
from jax import lax
from jax.experimental import pallas as pl
from jax.experimental.pallas import tpu as pltpu
from jax.experimental.pallas import tpu_sc as plsc

_NC = 2
_NS = 16
_CH = 80


def _make_agg(N, E, D):
    NW = _NC * _NS
    assert E % NW == 0 and (E // NW) % _CH == 0 and N % _NS == 0
    ept = E // NW
    nch = ept // _CH
    assert N % _CH == 0
    nrch = N // _CH
    mesh = plsc.VectorSubcoreMesh(core_axis_name="c", subcore_axis_name="s")

    assert nch % 2 == 1

    @functools.partial(
        pl.kernel,
        mesh=mesh,
        out_type=jax.ShapeDtypeStruct((_NC, N, D), jnp.float32),
        scratch_types=[
            pltpu.VMEM((ept,), jnp.int32),
            pltpu.VMEM((nch, _CH), jnp.int32),
            pltpu.VMEM((_CH, D), jnp.float32),
            pltpu.VMEM((_CH, D), jnp.float32),
            pltpu.VMEM_SHARED((N, D), jnp.float32),
            pltpu.SemaphoreType.DMA,
            pltpu.SemaphoreType.DMA,
            pltpu.SemaphoreType.DMA,
            pltpu.SemaphoreType.DMA,
        ],
    )
    def agg(x_hbm, src_hbm, dst_hbm, out_hbm, src_v, dst_v, rows0, rows1,
            acc_sh, sem0, sem1, ss0, ss1):
        c = lax.axis_index("c")
        s = lax.axis_index("s")
        wid = s * _NC + c

        idx_cp0 = pltpu.async_copy(src_hbm.at[pl.ds(wid * ept, ept)],
                                   src_v, sem0)
        idx_cp1 = pltpu.async_copy(dst_hbm.at[wid], dst_v, sem1)

        zero16 = jnp.zeros((16,), jnp.float32)

        def zfill(i, _):
            rows0[i // (D // 16), pl.ds((i % (D // 16)) * 16, 16)] = zero16
            return 0

        lax.fori_loop(0, _CH * D // 16, zfill, 0)

        nz = (nrch + _NS - 1) // _NS

        def zfire(j, _):
            k = s + j * _NS

            @pl.when(k < nrch)
            def _():
                pltpu.async_copy(rows0, acc_sh.at[pl.ds(k * _CH, _CH)], ss0)

            return 0

        def zdrain(j, _):
            k = s + j * _NS

            @pl.when(k < nrch)
            def _():
                pltpu.make_async_copy(
                    rows0, acc_sh.at[pl.ds(k * _CH, _CH)], ss0).wait()

            return 0

        lax.fori_loop(0, nz, zfire, 0)
        lax.fori_loop(0, nz, zdrain, 0)
        idx_cp0.wait()
        idx_cp1.wait()
        plsc.subcore_barrier()

        def sidx(j):
            return src_v.at[pl.ds(j * _CH, _CH)]

        pltpu.async_copy(x_hbm.at[sidx(0)], rows0, sem0)
        pltpu.async_copy(x_hbm.at[sidx(1)], rows1, sem1)

        def pair(i, _):
            j = 2 * i
            pltpu.make_async_copy(x_hbm.at[sidx(j)], rows0, sem0).wait()
            pltpu.async_copy(rows0, acc_sh.at[dst_v.at[j]], ss0, add=True)
            pltpu.make_async_copy(x_hbm.at[sidx(j + 1)], rows1, sem1).wait()
            pltpu.async_copy(rows1, acc_sh.at[dst_v.at[j + 1]], ss1, add=True)
            pltpu.make_async_copy(rows0, acc_sh.at[dst_v.at[j]], ss0).wait()
            pltpu.async_copy(x_hbm.at[sidx(j + 2)], rows0, sem0)
            pltpu.make_async_copy(rows1, acc_sh.at[dst_v.at[j + 1]],
                                  ss1).wait()

            @pl.when(j + 3 < nch)
            def _():
                pltpu.async_copy(x_hbm.at[sidx(j + 3)], rows1, sem1)

            return 0

        lax.fori_loop(0, (nch - 1) // 2, pair, 0)
        pltpu.make_async_copy(x_hbm.at[sidx(nch - 1)], rows0, sem0).wait()
        pltpu.sync_copy(rows0, acc_sh.at[dst_v.at[nch - 1]], add=True)
        plsc.subcore_barrier()

        bufs = (rows0, rows1)
        lsems = (sem0, sem1)
        wsems = (ss0, ss1)
        for j in range(nz):
            k = s + j * _NS
            b, ls, ws = bufs[j % 2], lsems[j % 2], wsems[j % 2]

            if j >= 2:

                @pl.when(k - 2 * _NS < nrch)
                def _(k=k, b=b, ws=ws):
                    rp = (k - 2 * _NS) * _CH
                    pltpu.make_async_copy(
                        b, out_hbm.at[c, pl.ds(rp, _CH)], ws).wait()

            @pl.when(k < nrch)
            def _(k=k, b=b, ls=ls, ws=ws):
                rr = k * _CH
                pltpu.async_copy(acc_sh.at[pl.ds(rr, _CH)], b, ls)
                pltpu.make_async_copy(
                    acc_sh.at[pl.ds(rr, _CH)], b, ls).wait()
                pltpu.async_copy(b, out_hbm.at[c, pl.ds(rr, _CH)], ws)

        for j in range(max(0, nz - 2), nz):
            k = s + j * _NS
            b, ws = bufs[j % 2], wsems[j % 2]

            @pl.when(k < nrch)
            def _(k=k, b=b, ws=ws):
                pltpu.make_async_copy(
                    b, out_hbm.at[c, pl.ds(k * _CH, _CH)], ws).wait()

    return agg


def _final(x, p0, p1, deg_inv, W):
    N, D = x.shape
    DO = W.shape[1]
    B = 2000
    assert N % B == 0

    def body(x_ref, p0_ref, p1_ref, dinv_ref, w_ref, o_ref):
        az = x_ref[...] + p0_ref[...] + p1_ref[...]
        azw = jnp.dot(az, w_ref[...], preferred_element_type=jnp.float32)
        o_ref[...] = jnp.tanh(dinv_ref[...] * azw)

    return pl.pallas_call(
        body,
        grid=(N // B,),
        in_specs=[
            pl.BlockSpec((B, D), lambda i: (i, 0)),
            pl.BlockSpec((B, D), lambda i: (i, 0)),
            pl.BlockSpec((B, D), lambda i: (i, 0)),
            pl.BlockSpec((B, 1), lambda i: (i, 0)),
            pl.BlockSpec((D, DO), lambda i: (0, 0)),
        ],
        out_specs=pl.BlockSpec((B, DO), lambda i: (i, 0)),
        out_shape=jax.ShapeDtypeStruct((N, DO), jnp.float32),
    )(x, p0, p1, deg_inv.reshape(N, 1), W)


def kernel(x, edge_index, deg_inv, W):
    N, D = x.shape
    E = edge_index.shape[1]
    NW = _NC * _NS
    nch = E // NW // _CH
    dst3 = edge_index[0].reshape(NW, nch, _CH)
    parts = _make_agg(N, E, D)(x, edge_index[1], dst3)
    return _final(x, parts[0], parts[1], deg_inv, W)
